# baseline (speedup 1.0000x reference)
import functools
import math

import jax
import jax.numpy as jnp
from jax import lax
from jax.experimental import pallas as pl
from jax.experimental.pallas import tpu as pltpu
from jax.experimental.pallas import tpu_sc as plsc

_NC = 2
_NS = 16
_NW = _NC * _NS
_LANES = 16


def _make_embed(batch: int, seq: int, d: int):
    assert batch % (128 * _NW) == 0 and batch // 128 == _NW
    assert seq % 8 == 0 and d % _LANES == 0
    n_trows = seq // 8
    seq_pad = ((seq + _LANES - 1) // _LANES) * _LANES
    n_sg = seq_pad // _LANES
    scale = jnp.float32(math.sqrt(d))
    mesh = plsc.VectorSubcoreMesh(core_axis_name="c", subcore_axis_name="s")

    @functools.partial(
        pl.kernel,
        mesh=mesh,
        out_type=jax.ShapeDtypeStruct((batch * seq, d), jnp.float32),
        scratch_types=[
            pltpu.VMEM((n_trows, 1024), jnp.int32),
            pltpu.VMEM((128, seq_pad), jnp.int32),
            pltpu.VMEM((seq_pad, d), jnp.float32),
            pltpu.VMEM((seq_pad, d), jnp.float32),
            pltpu.SemaphoreType.DMA,
            pltpu.SemaphoreType.DMA,
        ],
        compiler_params=pltpu.CompilerParams(
            use_tc_tiling_on_sc=False, needs_layout_passes=False
        ),
    )
    def embed(idx_hbm, table_hbm, out_hbm, idx_all, idx_t, rows0, rows1,
              sem0, sem1):
        wid = lax.axis_index("s") * _NC + lax.axis_index("c")
        pltpu.sync_copy(idx_hbm.at[:, wid], idx_all)

        lanes = lax.iota(jnp.int32, 16)
        row_v = lanes >> 3
        col_v = (lanes & 7) * 128

        def build_b(b, carry):
            for sg in range(n_sg):
                r = jnp.minimum(row_v + 2 * sg, n_trows - 1)
                c = col_v + b
                g = plsc.load_gather(idx_all, [r, c])
                if (sg + 1) * _LANES > seq:
                    g = jnp.where(sg * _LANES + lanes < seq, g, 0)
                idx_t[b, pl.ds(sg * _LANES, _LANES)] = g
            return carry

        lax.fori_loop(0, 128, build_b, 0)

        def start_gather(b, rows_v, sem):
            pltpu.async_copy(table_hbm.at[idx_t.at[b]], rows_v, sem)

        def finish_b(b, rows_v, sem):
            pltpu.make_async_copy(table_hbm.at[idx_t.at[b]], rows_v, sem).wait()

            def row_body(r, carry2):
                for j in range(d // _LANES):
                    sl = pl.ds(j * _LANES, _LANES)
                    rows_v[r, sl] = rows_v[r, sl] * scale
                return carry2

            lax.fori_loop(0, seq_pad, row_body, 0, unroll=2)
            pltpu.sync_copy(
                rows_v.at[pl.ds(0, seq)],
                out_hbm.at[pl.ds((wid * 128 + b) * seq, seq)],
            )

        start_gather(0, rows0, sem0)

        def pair_body(p, carry):
            b = 2 * p
            start_gather(b + 1, rows1, sem1)
            finish_b(b, rows0, sem0)

            @pl.when(p + 1 < 64)
            def _():
                start_gather(b + 2, rows0, sem0)

            finish_b(b + 1, rows1, sem1)
            return carry

        lax.fori_loop(0, 64, pair_body, 0)

    return embed


def kernel(x, table):
    b, s = x.shape
    vocab, d = table.shape
    x4 = (
        x.T.reshape(s // 8, 8, b // 128, 128)
        .transpose(0, 2, 1, 3)
        .reshape(s // 8, b // 128, 1024)
    )
    rows = _make_embed(b, s, d)(x4, table)
    return rows.reshape(b, s, d)
